# Initial kernel scaffold; baseline (speedup 1.0000x reference)
import jax
import jax.numpy as jnp
from jax.experimental import pallas as pl


def kernel(x, pos_table, ln_weight, ln_bias):
    raise NotImplementedError("write your pallas kernel here")


# speedup vs baseline: 2.6120x; 2.6120x over previous
import functools

import jax
import jax.numpy as jnp
from jax.experimental import pallas as pl
from jax.experimental.pallas import tpu as pltpu


_EPS = 1e-12
_BB = 16


def _tc_body(x_ref, pos_ref, w_ref, b_ref, o_ref):
    e = x_ref[...] + pos_ref[...] * 0.1
    u = jnp.mean(e, axis=-1, keepdims=True)
    c = e - u
    s = jnp.mean(c * c, axis=-1, keepdims=True)
    o_ref[...] = w_ref[...] * (c * jax.lax.rsqrt(s + _EPS)) + b_ref[...]


@jax.jit
def kernel(x, pos_table, ln_weight, ln_bias):
    bz, seq, d = x.shape
    pos = pos_table[:seq]
    w = ln_weight.reshape(1, 1, d)
    b = ln_bias.reshape(1, 1, d)
    grid = (bz // _BB,)
    return pl.pallas_call(
        _tc_body,
        grid=grid,
        in_specs=[
            pl.BlockSpec((_BB, seq, d), lambda i: (i, 0, 0)),
            pl.BlockSpec((seq, d), lambda i: (0, 0)),
            pl.BlockSpec((1, 1, d), lambda i: (0, 0, 0)),
            pl.BlockSpec((1, 1, d), lambda i: (0, 0, 0)),
        ],
        out_specs=pl.BlockSpec((_BB, seq, d), lambda i: (i, 0, 0)),
        out_shape=jax.ShapeDtypeStruct((bz, seq, d), x.dtype),
    )(x, pos, w, b)
